# baseline (speedup 1.0000x reference)
import jax
import jax.numpy as jnp
from jax import lax
from jax.experimental import pallas as pl
from jax.experimental.pallas import tpu as pltpu
from jax.experimental.pallas import tpu_sc as plsc

N_NODES = 100000
N_SPECIES = 119
EMBED_DIM = 256
NC = 2
NS = 16
NW = NC * NS

LANES = 16
CHUNK = 16
IDX_OFF = 8

ROWS_MAIN = 3136
ROWS_LAST = N_NODES - 31 * ROWS_MAIN
N_CHUNKS = ROWS_MAIN // CHUNK


NBUF = 4


def _gather_body(idx_hbm, table_hbm, out_hbm,
                 table_v, idx0, idx1, idx2, idx3, rows0, rows1, rows2, rows3,
                 tsem, isem0, isem1, isem2, isem3,
                 osem0, osem1, osem2, osem3):
    wid = lax.axis_index("s") * NC + lax.axis_index("c")
    base = wid * ROWS_MAIN
    count = jnp.where(wid == NW - 1, ROWS_LAST, ROWS_MAIN)
    last_start = base + count - CHUNK

    idx_bufs = (idx0, idx1, idx2, idx3)
    rows_bufs = (rows0, rows1, rows2, rows3)
    isems = (isem0, isem1, isem2, isem3)
    osems = (osem0, osem1, osem2, osem3)

    lane_iota = jax.lax.iota(jnp.int32, LANES)
    cols = [lane_iota + c * LANES for c in range(EMBED_DIM // LANES)]

    def cstart(j):
        return jnp.minimum(base + j * CHUNK, last_start)

    def idx_copy(j, b):
        return pltpu.make_async_copy(
            idx_hbm.at[pl.ds(cstart(j), CHUNK)],
            idx_bufs[b].at[pl.ds(IDX_OFF, CHUNK)], isems[b])

    def store_copy(j, b):
        return pltpu.make_async_copy(
            rows_bufs[b], out_hbm.at[pl.ds(cstart(j), CHUNK)], osems[b])

    def compute(b):
        rows = [plsc.load_gather(
                    idx_bufs[b], [jnp.full((LANES,), IDX_OFF + r, jnp.int32)])
                for r in range(CHUNK)]
        rowbases = [lax.shift_left(row, jnp.int32(8)) for row in rows]
        for r in range(CHUNK):
            xs = [plsc.load_gather(table_v, [rowbases[r] + cols[c]])
                  for c in range(EMBED_DIM // LANES)]
            for c in range(EMBED_DIM // LANES):
                rows_bufs[b][r, pl.ds(c * LANES, LANES)] = xs[c]

    tcp = pltpu.make_async_copy(table_hbm, table_v, tsem)
    tcp.start()
    for b in range(NBUF):
        idx_copy(b, b).start()
    tcp.wait()

    for b in range(NBUF):
        idx_copy(b, b).wait()
        compute(b)
        store_copy(b, b).start()
        idx_copy(b + NBUF, b).start()

    def body(p, carry):
        for b in range(NBUF):
            j = NBUF * p + b
            idx_copy(j, b).wait()
            store_copy(j - NBUF, b).wait()
            compute(b)
            store_copy(j, b).start()
            idx_copy(j + NBUF, b).start()
        return carry

    lax.fori_loop(1, N_CHUNKS // NBUF, body, None)

    for b in range(NBUF):
        idx_copy(N_CHUNKS + b, b).wait()
        store_copy(N_CHUNKS - NBUF + b, b).wait()


@jax.jit
def _gather(node_specie, embeddings_flat):
    mesh = plsc.VectorSubcoreMesh(
        core_axis_name="c", subcore_axis_name="s",
        num_cores=NC, num_subcores=NS)
    return pl.kernel(
        _gather_body,
        out_type=jax.ShapeDtypeStruct((N_NODES, EMBED_DIM), jnp.float32),
        mesh=mesh,
        compiler_params=pltpu.CompilerParams(needs_layout_passes=False),
        scratch_types=(
            [pltpu.VMEM((N_SPECIES * EMBED_DIM,), jnp.float32)]
            + [pltpu.VMEM((IDX_OFF + CHUNK,), jnp.int32)] * NBUF
            + [pltpu.VMEM((CHUNK, EMBED_DIM), jnp.float32)] * NBUF
            + [pltpu.SemaphoreType.DMA] * (1 + 2 * NBUF)
        ),
        name="embedding_gather_sc",
    )(node_specie, embeddings_flat)


def kernel(node_specie, embeddings):
    return _gather(node_specie.astype(jnp.int32),
                   embeddings.reshape(N_SPECIES * EMBED_DIM))


# speedup vs baseline: 1.7696x; 1.7696x over previous
import jax
import jax.numpy as jnp
from jax import lax
from jax.experimental import pallas as pl
from jax.experimental.pallas import tpu as pltpu
from jax.experimental.pallas import tpu_sc as plsc

N_NODES = 100000
N_SPECIES = 119
EMBED_DIM = 256
NC = 2
NS = 16
NW = NC * NS

LANES = 16
CHUNK = 16
IDX_OFF = 8

ROWS_MAIN = 3136
ROWS_LAST = N_NODES - 31 * ROWS_MAIN
N_CHUNKS = ROWS_MAIN // CHUNK


NBUF = 2


def _gather_body(idx_hbm, table_hbm, out_hbm,
                 table_v, idx0, idx1, rows0, rows1,
                 tsem, isem0, isem1, osem0, osem1):
    wid = lax.axis_index("s") * NC + lax.axis_index("c")
    base = wid * ROWS_MAIN
    count = jnp.where(wid == NW - 1, ROWS_LAST, ROWS_MAIN)
    last_start = base + count - CHUNK

    idx_bufs = (idx0, idx1)
    rows_bufs = (rows0, rows1)
    isems = (isem0, isem1)
    osems = (osem0, osem1)

    lane_iota = jax.lax.iota(jnp.int32, LANES)
    cols = [lane_iota + c * LANES for c in range(EMBED_DIM // LANES)]

    def cstart(j):
        return jnp.minimum(base + j * CHUNK, last_start)

    def idx_copy(j, b):
        return pltpu.make_async_copy(
            idx_hbm.at[pl.ds(cstart(j), CHUNK)],
            idx_bufs[b].at[pl.ds(IDX_OFF, CHUNK)], isems[b])

    def store_copy(j, b):
        return pltpu.make_async_copy(
            rows_bufs[b], out_hbm.at[pl.ds(cstart(j), CHUNK)], osems[b])

    def compute(b):
        rows = [plsc.load_gather(
                    idx_bufs[b], [jnp.full((LANES,), IDX_OFF + r, jnp.int32)])
                for r in range(CHUNK)]
        rowbases = [lax.shift_left(row, jnp.int32(8)) for row in rows]
        for r in range(CHUNK):
            xs = [plsc.load_gather(table_v, [rowbases[r] + cols[c]])
                  for c in range(EMBED_DIM // LANES)]
            for c in range(EMBED_DIM // LANES):
                rows_bufs[b][r, pl.ds(c * LANES, LANES)] = xs[c]

    tcp = pltpu.make_async_copy(table_hbm, table_v, tsem)
    tcp.start()
    for b in range(NBUF):
        idx_copy(b, b).start()
    tcp.wait()

    for b in range(NBUF):
        idx_copy(b, b).wait()
        compute(b)
        store_copy(b, b).start()
        idx_copy(b + NBUF, b).start()

    def body(p, carry):
        for b in range(NBUF):
            j = NBUF * p + b
            idx_copy(j, b).wait()
            store_copy(j - NBUF, b).wait()
            compute(b)
            store_copy(j, b).start()
            idx_copy(j + NBUF, b).start()
        return carry

    lax.fori_loop(1, N_CHUNKS // NBUF, body, None)

    for b in range(NBUF):
        idx_copy(N_CHUNKS + b, b).wait()
        store_copy(N_CHUNKS - NBUF + b, b).wait()


@jax.jit
def _gather(node_specie, embeddings_flat):
    mesh = plsc.VectorSubcoreMesh(
        core_axis_name="c", subcore_axis_name="s",
        num_cores=NC, num_subcores=NS)
    return pl.kernel(
        _gather_body,
        out_type=jax.ShapeDtypeStruct((N_NODES, EMBED_DIM), jnp.float32),
        mesh=mesh,
        compiler_params=pltpu.CompilerParams(needs_layout_passes=False),
        scratch_types=(
            [pltpu.VMEM((N_SPECIES * EMBED_DIM,), jnp.float32)]
            + [pltpu.VMEM((IDX_OFF + CHUNK,), jnp.int32)] * NBUF
            + [pltpu.VMEM((CHUNK, EMBED_DIM), jnp.float32)] * NBUF
            + [pltpu.SemaphoreType.DMA] * (1 + 2 * NBUF)
        ),
        name="embedding_gather_sc",
    )(node_specie, embeddings_flat)


def kernel(node_specie, embeddings):
    return _gather(node_specie.astype(jnp.int32),
                   embeddings.reshape(N_SPECIES * EMBED_DIM))
